# baseline (device time: 97696 ns/iter reference)
import functools

import jax
import jax.numpy as jnp
from jax import lax
from jax.experimental import pallas as pl
from jax.experimental.pallas import tpu as pltpu

N_DEV = 4
SQ = 256
SKV_LOCAL = 4096
HQ = 8
DH = 128
DM = 1024
SCALE = 0.08838834764831843
NEG = -1e9


def _attn_body(x_ref, wq_ref, k_ref, v_ref, o_ref, st_ref, bias_ref):
    h = pl.program_id(0)

    @pl.when(h == 0)
    def _():
        my = lax.axis_index("i")
        qb = lax.broadcasted_iota(jnp.int32, (SQ, SKV_LOCAL), 0) // 64
        kb = lax.broadcasted_iota(jnp.int32, (SQ, SKV_LOCAL), 1) // 64 + my * 64
        mask = (qb == kb) | (kb == 0) | (((qb + kb) % 3) == 0)
        bias_ref[...] = jnp.where(mask, 0.0, NEG).astype(jnp.float32)

    q = jnp.dot(
        x_ref[0].astype(jnp.bfloat16),
        wq_ref[...].astype(jnp.bfloat16),
        preferred_element_type=jnp.float32,
    )
    k = k_ref[0, :, h, :].astype(jnp.bfloat16)
    s = lax.dot_general(
        q.astype(jnp.bfloat16),
        k,
        ((((1,), (1,))), ((), ())),
        preferred_element_type=jnp.float32,
    )
    s = s * SCALE + bias_ref[...]
    m = jnp.max(s, axis=1, keepdims=True)
    w = jnp.exp(s - m)
    lsum = jnp.sum(w, axis=1, keepdims=True)
    o = lax.dot_general(
        w.astype(jnp.bfloat16),
        v_ref[0, :, h, :].astype(jnp.bfloat16),
        ((((1,), (0,))), ((), ())),
        preferred_element_type=jnp.float32,
    )
    o_ref[0] = o.astype(jnp.bfloat16)
    st_ref[0, 0, :] = m[:, 0]
    st_ref[0, 1, :] = lsum[:, 0]


def _merge_body(
    o_ref,
    st_ref,
    wo_ref,
    out_ref,
    comm_o,
    comm_st,
    acc_o,
    acc_st,
    ctx,
    send_o_sem,
    recv_o_sem,
    send_st_sem,
    recv_st_sem,
):
    my = lax.axis_index("i")
    partners = [my ^ 1, my ^ 2]

    barrier_sem = pltpu.get_barrier_semaphore()
    for p in partners:
        pl.semaphore_signal(
            barrier_sem, inc=1, device_id=(p,),
            device_id_type=pl.DeviceIdType.MESH,
        )
    pl.semaphore_wait(barrier_sem, 2)

    for r in range(2):
        p = partners[r]
        src_o = o_ref if r == 0 else acc_o
        src_st = st_ref if r == 0 else acc_st
        rd_o = pltpu.make_async_remote_copy(
            src_ref=src_o,
            dst_ref=comm_o.at[r],
            send_sem=send_o_sem.at[r],
            recv_sem=recv_o_sem.at[r],
            device_id=(p,),
            device_id_type=pl.DeviceIdType.MESH,
        )
        rd_st = pltpu.make_async_remote_copy(
            src_ref=src_st,
            dst_ref=comm_st.at[r],
            send_sem=send_st_sem.at[r],
            recv_sem=recv_st_sem.at[r],
            device_id=(p,),
            device_id_type=pl.DeviceIdType.MESH,
        )
        rd_o.start()
        rd_st.start()
        rd_o.wait()
        rd_st.wait()

        m_a = src_st[:, 0, :]
        l_a = src_st[:, 1, :]
        m_b = comm_st[r, :, 0, :]
        l_b = comm_st[r, :, 1, :]
        mx = jnp.maximum(m_a, m_b)
        sa = jnp.exp(m_a - mx)
        sb = jnp.exp(m_b - mx)
        merged = (
            src_o[...].astype(jnp.float32) * sa[:, :, None]
            + comm_o[r].astype(jnp.float32) * sb[:, :, None]
        )
        acc_o[...] = merged.astype(jnp.bfloat16)
        acc_st[:, 0, :] = mx
        acc_st[:, 1, :] = l_a * sa + l_b * sb

    lsum = acc_st[:, 1, :]
    ctxv = acc_o[...].astype(jnp.float32) / lsum[:, :, None]
    for hh in range(HQ):
        ctx[:, hh * DH:(hh + 1) * DH] = ctxv[hh].astype(jnp.bfloat16)
    out_ref[0] = jnp.dot(
        ctx[...],
        wo_ref[...].astype(jnp.bfloat16),
        preferred_element_type=jnp.float32,
    )


def kernel(x, Wq, K_ext, V_ext, Wo):
    o, stats = pl.pallas_call(
        _attn_body,
        grid=(HQ,),
        in_specs=[
            pl.BlockSpec((1, SQ, DM), lambda h: (0, 0, 0)),
            pl.BlockSpec((DM, DH), lambda h: (0, h)),
            pl.BlockSpec((1, SKV_LOCAL, HQ, DH), lambda h: (0, 0, 0, 0)),
            pl.BlockSpec((1, SKV_LOCAL, HQ, DH), lambda h: (0, 0, 0, 0)),
        ],
        out_shape=[
            jax.ShapeDtypeStruct((HQ, SQ, DH), jnp.bfloat16),
            jax.ShapeDtypeStruct((HQ, 2, SQ), jnp.float32),
        ],
        out_specs=[
            pl.BlockSpec((1, SQ, DH), lambda h: (h, 0, 0)),
            pl.BlockSpec((1, 2, SQ), lambda h: (h, 0, 0)),
        ],
        scratch_shapes=[pltpu.VMEM((SQ, SKV_LOCAL), jnp.float32)],
        compiler_params=pltpu.CompilerParams(
            dimension_semantics=("arbitrary",),
        ),
    )(x, Wq, K_ext, V_ext)

    out = pl.pallas_call(
        _merge_body,
        in_specs=[
            pl.BlockSpec(memory_space=pltpu.VMEM),
            pl.BlockSpec(memory_space=pltpu.VMEM),
            pl.BlockSpec(memory_space=pltpu.VMEM),
        ],
        out_shape=jax.ShapeDtypeStruct((1, SQ, DM), jnp.float32),
        out_specs=pl.BlockSpec(memory_space=pltpu.VMEM),
        scratch_shapes=[
            pltpu.VMEM((2, HQ, SQ, DH), jnp.bfloat16),
            pltpu.VMEM((2, HQ, 2, SQ), jnp.float32),
            pltpu.VMEM((HQ, SQ, DH), jnp.bfloat16),
            pltpu.VMEM((HQ, 2, SQ), jnp.float32),
            pltpu.VMEM((SQ, DM), jnp.bfloat16),
            pltpu.SemaphoreType.DMA((2,)),
            pltpu.SemaphoreType.DMA((2,)),
            pltpu.SemaphoreType.DMA((2,)),
            pltpu.SemaphoreType.DMA((2,)),
        ],
        compiler_params=pltpu.CompilerParams(collective_id=0),
    )(o, stats, Wo)
    return out


# device time: 68920 ns/iter; 1.4175x vs baseline; 1.4175x over previous
import functools

import jax
import jax.numpy as jnp
from jax import lax
from jax.experimental import pallas as pl
from jax.experimental.pallas import tpu as pltpu

N_DEV = 4
SQ = 256
SKV_LOCAL = 4096
HQ = 8
DH = 128
DM = 1024
SCALE = 0.08838834764831843
NEG = -1e9


def _head_copies(k_hbm, v_hbm, kbuf, vbuf, ksem, vsem, head, slot):
    ck = pltpu.make_async_copy(
        k_hbm.at[0, :, head, :], kbuf.at[slot], ksem.at[slot]
    )
    cv = pltpu.make_async_copy(
        v_hbm.at[0, :, head, :], vbuf.at[slot], vsem.at[slot]
    )
    return ck, cv


def _attn_body(
    x_ref, wq_ref, k_hbm, v_hbm, o_ref, st_ref,
    bias_ref, kbuf, vbuf, ksem, vsem,
):
    h = pl.program_id(0)
    slot = h % 2
    nslot = (h + 1) % 2

    @pl.when(h == 0)
    def _():
        ck, cv = _head_copies(k_hbm, v_hbm, kbuf, vbuf, ksem, vsem, 0, 0)
        ck.start()
        cv.start()
        my = lax.axis_index("i")
        qb = lax.broadcasted_iota(jnp.int32, (SQ, SKV_LOCAL), 0) // 64
        kb = lax.broadcasted_iota(jnp.int32, (SQ, SKV_LOCAL), 1) // 64 + my * 64
        mask = (qb == kb) | (kb == 0) | (((qb + kb) % 3) == 0)
        bias_ref[...] = jnp.where(mask, 0.0, NEG).astype(jnp.float32)

    @pl.when(h + 1 < HQ)
    def _():
        ck, cv = _head_copies(
            k_hbm, v_hbm, kbuf, vbuf, ksem, vsem, h + 1, nslot
        )
        ck.start()
        cv.start()

    q = jnp.dot(
        x_ref[0].astype(jnp.bfloat16),
        wq_ref[...].astype(jnp.bfloat16),
        preferred_element_type=jnp.float32,
    )
    ck, cv = _head_copies(k_hbm, v_hbm, kbuf, vbuf, ksem, vsem, h, slot)
    ck.wait()
    k = kbuf[slot].astype(jnp.bfloat16)
    s = lax.dot_general(
        q.astype(jnp.bfloat16),
        k,
        ((((1,), (1,))), ((), ())),
        preferred_element_type=jnp.float32,
    )
    s = s * SCALE + bias_ref[...]
    m = jnp.max(s, axis=1, keepdims=True)
    w = jnp.exp(s - m)
    lsum = jnp.sum(w, axis=1, keepdims=True)
    cv.wait()
    o = lax.dot_general(
        w.astype(jnp.bfloat16),
        vbuf[slot].astype(jnp.bfloat16),
        ((((1,), (0,))), ((), ())),
        preferred_element_type=jnp.float32,
    )
    o_ref[0] = o.astype(jnp.bfloat16)
    st_ref[0, 0, :] = m[:, 0]
    st_ref[0, 1, :] = lsum[:, 0]


def _merge_body(
    o_ref,
    st_ref,
    wo_ref,
    out_ref,
    comm_o,
    comm_st,
    acc_o,
    acc_st,
    ctx,
    send_o_sem,
    recv_o_sem,
    send_st_sem,
    recv_st_sem,
):
    my = lax.axis_index("i")
    partners = [my ^ 1, my ^ 2]

    barrier_sem = pltpu.get_barrier_semaphore()
    for p in partners:
        pl.semaphore_signal(
            barrier_sem, inc=1, device_id=(p,),
            device_id_type=pl.DeviceIdType.MESH,
        )
    pl.semaphore_wait(barrier_sem, 2)

    for r in range(2):
        p = partners[r]
        src_o = o_ref if r == 0 else acc_o
        src_st = st_ref if r == 0 else acc_st
        rd_o = pltpu.make_async_remote_copy(
            src_ref=src_o,
            dst_ref=comm_o.at[r],
            send_sem=send_o_sem.at[r],
            recv_sem=recv_o_sem.at[r],
            device_id=(p,),
            device_id_type=pl.DeviceIdType.MESH,
        )
        rd_st = pltpu.make_async_remote_copy(
            src_ref=src_st,
            dst_ref=comm_st.at[r],
            send_sem=send_st_sem.at[r],
            recv_sem=recv_st_sem.at[r],
            device_id=(p,),
            device_id_type=pl.DeviceIdType.MESH,
        )
        rd_o.start()
        rd_st.start()
        rd_o.wait()
        rd_st.wait()

        m_a = src_st[:, 0, :]
        l_a = src_st[:, 1, :]
        m_b = comm_st[r, :, 0, :]
        l_b = comm_st[r, :, 1, :]
        mx = jnp.maximum(m_a, m_b)
        sa = jnp.exp(m_a - mx)
        sb = jnp.exp(m_b - mx)
        merged = (
            src_o[...].astype(jnp.float32) * sa[:, :, None]
            + comm_o[r].astype(jnp.float32) * sb[:, :, None]
        )
        acc_o[...] = merged.astype(jnp.bfloat16)
        acc_st[:, 0, :] = mx
        acc_st[:, 1, :] = l_a * sa + l_b * sb

    lsum = acc_st[:, 1, :]
    ctxv = acc_o[...].astype(jnp.float32) / lsum[:, :, None]
    for hh in range(HQ):
        ctx[:, hh * DH:(hh + 1) * DH] = ctxv[hh].astype(jnp.bfloat16)
    out_ref[0] = jnp.dot(
        ctx[...],
        wo_ref[...].astype(jnp.bfloat16),
        preferred_element_type=jnp.float32,
    )


def kernel(x, Wq, K_ext, V_ext, Wo):
    o, stats = pl.pallas_call(
        _attn_body,
        grid=(HQ,),
        in_specs=[
            pl.BlockSpec((1, SQ, DM), lambda h: (0, 0, 0)),
            pl.BlockSpec((DM, DH), lambda h: (0, h)),
            pl.BlockSpec(memory_space=pl.ANY),
            pl.BlockSpec(memory_space=pl.ANY),
        ],
        out_shape=[
            jax.ShapeDtypeStruct((HQ, SQ, DH), jnp.bfloat16),
            jax.ShapeDtypeStruct((HQ, 2, SQ), jnp.float32),
        ],
        out_specs=[
            pl.BlockSpec((1, SQ, DH), lambda h: (h, 0, 0)),
            pl.BlockSpec((1, 2, SQ), lambda h: (h, 0, 0)),
        ],
        scratch_shapes=[
            pltpu.VMEM((SQ, SKV_LOCAL), jnp.float32),
            pltpu.VMEM((2, SKV_LOCAL, DH), jnp.float32),
            pltpu.VMEM((2, SKV_LOCAL, DH), jnp.float32),
            pltpu.SemaphoreType.DMA((2,)),
            pltpu.SemaphoreType.DMA((2,)),
        ],
        compiler_params=pltpu.CompilerParams(
            dimension_semantics=("arbitrary",),
        ),
    )(x, Wq, K_ext, V_ext)

    out = pl.pallas_call(
        _merge_body,
        in_specs=[
            pl.BlockSpec(memory_space=pltpu.VMEM),
            pl.BlockSpec(memory_space=pltpu.VMEM),
            pl.BlockSpec(memory_space=pltpu.VMEM),
        ],
        out_shape=jax.ShapeDtypeStruct((1, SQ, DM), jnp.float32),
        out_specs=pl.BlockSpec(memory_space=pltpu.VMEM),
        scratch_shapes=[
            pltpu.VMEM((2, HQ, SQ, DH), jnp.bfloat16),
            pltpu.VMEM((2, HQ, 2, SQ), jnp.float32),
            pltpu.VMEM((HQ, SQ, DH), jnp.bfloat16),
            pltpu.VMEM((HQ, 2, SQ), jnp.float32),
            pltpu.VMEM((SQ, DM), jnp.bfloat16),
            pltpu.SemaphoreType.DMA((2,)),
            pltpu.SemaphoreType.DMA((2,)),
            pltpu.SemaphoreType.DMA((2,)),
            pltpu.SemaphoreType.DMA((2,)),
        ],
        compiler_params=pltpu.CompilerParams(collective_id=0),
    )(o, stats, Wo)
    return out
